# Initial kernel scaffold; baseline (speedup 1.0000x reference)
import jax
import jax.numpy as jnp
from jax.experimental import pallas as pl


def kernel(x, edge_index, edge_type, batch, W_rel1, root1, b1, W_rel2, root2, b2, linW, linb):
    raise NotImplementedError("write your pallas kernel here")


# speedup vs baseline: 20.0608x; 20.0608x over previous
import functools

import jax
import jax.numpy as jnp
from jax import lax
from jax.experimental import pallas as pl
from jax.experimental.pallas import tpu as pltpu
from jax.experimental.pallas import tpu_sc as plsc

N = 10000
E = 320000
D = 128
HID = 128
R = 8
C = 16
G = 64

NC = 2
NS = 16
NW = NC * NS

B = 80
EPT = E // NW
EPS = E // NS
DEGP = 81920
ZCH = DEGP // NS
NPAD = 10240
ROWS_T = NPAD // NS
CH = 2000

_mesh = plsc.VectorSubcoreMesh(core_axis_name="c", subcore_axis_name="s")


def _f32(shape):
    return jax.ShapeDtypeStruct(shape, jnp.float32)


def _prep_body(type_h, src_h, dst_h, w_h, gidx_h,
               deg_sh, deg_v, tb, sb, db, comb_b, ones_b, zb, wb, gb, sem):
    sid = lax.axis_index("s")
    cid = lax.axis_index("c")
    wid = sid * NC + cid

    for i in range(32):
        zb[pl.ds(i * 16, 16)] = jnp.zeros((16,), jnp.float32)
    for i in range(5):
        ones_b[pl.ds(i * 16, 16)] = jnp.ones((16,), jnp.float32)

    def _zero(k, carry):
        pltpu.sync_copy(zb, deg_sh.at[pl.ds(sid * ZCH + k * 512, 512)])
        return carry
    lax.fori_loop(0, ZCH // 512, _zero, 0)
    plsc.subcore_barrier()

    base1 = sid * EPS

    def _count_round(r, carry):
        off = base1 + r * CH
        pltpu.sync_copy(type_h.at[pl.ds(off, CH)], tb)
        pltpu.sync_copy(dst_h.at[pl.ds(off, CH)], db)

        def _blk(b, c2):
            for g in range(5):
                t = tb[pl.ds(b * B + g * 16, 16)]
                dd = db[pl.ds(b * B + g * 16, 16)]
                comb_b[pl.ds(g * 16, 16)] = t * N + dd
            pltpu.sync_copy(ones_b, deg_sh.at[comb_b], add=True)
            return c2
        return lax.fori_loop(0, CH // B, _blk, carry)
    lax.fori_loop(0, EPS // CH, _count_round, 0)
    plsc.subcore_barrier()

    pltpu.sync_copy(deg_sh, deg_v)

    base2 = wid * EPT

    def _emit_round(r, carry):
        off = base2 + r * CH
        pltpu.sync_copy(type_h.at[pl.ds(off, CH)], tb)
        pltpu.sync_copy(src_h.at[pl.ds(off, CH)], sb)
        pltpu.sync_copy(dst_h.at[pl.ds(off, CH)], db)

        def _grp(g, c2):
            t = tb[pl.ds(g * 16, 16)]
            s = sb[pl.ds(g * 16, 16)]
            dd = db[pl.ds(g * 16, 16)]
            degv = plsc.load_gather(deg_v, [t * N + dd])
            wb[pl.ds(g * 16, 16)] = 1.0 / jnp.maximum(degv, 1.0)
            gb[pl.ds(g * 16, 16)] = t * N + s
            return c2
        lax.fori_loop(0, CH // 16, _grp, carry)
        pltpu.sync_copy(wb, w_h.at[pl.ds(off, CH)])
        pltpu.sync_copy(gb, gidx_h.at[pl.ds(off, CH)])
        return carry
    lax.fori_loop(0, EPT // CH, _emit_round, 0)


_prep = pl.kernel(
    _prep_body,
    out_type=(_f32((E,)), jax.ShapeDtypeStruct((E,), jnp.int32)),
    mesh=_mesh,
    compiler_params=pltpu.CompilerParams(needs_layout_passes=False),
    scratch_types=[
        pltpu.VMEM_SHARED((DEGP,), jnp.float32),
        pltpu.VMEM((DEGP,), jnp.float32),
        pltpu.VMEM((CH,), jnp.int32),
        pltpu.VMEM((CH,), jnp.int32),
        pltpu.VMEM((CH,), jnp.int32),
        pltpu.VMEM((B,), jnp.int32),
        pltpu.VMEM((B,), jnp.float32),
        pltpu.VMEM((512,), jnp.float32),
        pltpu.VMEM((CH,), jnp.float32),
        pltpu.VMEM((CH,), jnp.int32),
        pltpu.SemaphoreType.DMA,
    ],
)


def _layer_body(hall_h, gidx_h, dst2_h, w_h, parts_h,
                acc_sh, gi_v, w_v, dst_v, msg, sem):
    sid = lax.axis_index("s")
    cid = lax.axis_index("c")
    wid = sid * NC + cid
    nblk = EPT // B

    def _zr(i, carry):
        for c8 in range(8):
            msg[i, pl.ds(c8 * 16, 16)] = jnp.zeros((16,), jnp.float32)
        return carry
    lax.fori_loop(0, B, _zr, 0)

    def _zero(k, carry):
        pltpu.sync_copy(msg, acc_sh.at[pl.ds(sid * ROWS_T + k * B, B)])
        return carry
    lax.fori_loop(0, ROWS_T // B, _zero, 0)

    pltpu.sync_copy(gidx_h.at[pl.ds(wid * EPT, EPT)], gi_v)
    pltpu.sync_copy(w_h.at[pl.ds(wid * EPT, EPT)], w_v)
    pltpu.sync_copy(dst2_h.at[wid], dst_v)
    plsc.subcore_barrier()

    def _blk(b, carry):
        pltpu.async_copy(hall_h.at[gi_v.at[pl.ds(b * B, B)]], msg, sem).wait()

        def _row16(g, c2):
            wv = w_v[pl.ds(b * B + g * 16, 16)]
            base = g * 16
            for j in range(16):
                wj = wv[j]
                for c8 in range(8):
                    msg[base + j, pl.ds(c8 * 16, 16)] = (
                        msg[base + j, pl.ds(c8 * 16, 16)] * wj)
            return c2
        lax.fori_loop(0, B // 16, _row16, carry)
        pltpu.sync_copy(msg, acc_sh.at[dst_v.at[b]], add=True)
        return carry
    lax.fori_loop(0, nblk, _blk, 0)
    plsc.subcore_barrier()

    def _out(k, carry):
        row0 = sid * ROWS_T + k * B
        pltpu.sync_copy(acc_sh.at[pl.ds(row0, B)], msg)
        pltpu.sync_copy(msg, parts_h.at[cid, pl.ds(row0, B)])
        return carry
    lax.fori_loop(0, ROWS_T // B, _out, 0)


_layer_sc = pl.kernel(
    _layer_body,
    out_type=_f32((NC, NPAD, HID)),
    mesh=_mesh,
    scratch_types=[
        pltpu.VMEM_SHARED((NPAD, HID), jnp.float32),
        pltpu.VMEM((EPT,), jnp.int32),
        pltpu.VMEM((EPT,), jnp.float32),
        pltpu.VMEM((EPT // B, B), jnp.int32),
        pltpu.VMEM((B, HID), jnp.float32),
        pltpu.SemaphoreType.DMA,
    ],
)


BN = 1000


def _mm_body(x_ref, w_ref, o_ref):
    o_ref[0] = jnp.dot(x_ref[...], w_ref[0],
                       preferred_element_type=jnp.float32)


def _hall(x, W_rel):
    return pl.pallas_call(
        _mm_body,
        grid=(R, N // BN),
        in_specs=[
            pl.BlockSpec((BN, D), lambda r, n: (n, 0)),
            pl.BlockSpec((1, D, HID), lambda r, n: (r, 0, 0)),
        ],
        out_specs=pl.BlockSpec((1, BN, HID), lambda r, n: (r, n, 0)),
        out_shape=_f32((R, N, HID)),
    )(x, W_rel)


def _comb_body(x_ref, p0_ref, p1_ref, root_ref, b_ref, o_ref):
    agg = p0_ref[...] + p1_ref[...]
    z = agg + jnp.dot(x_ref[...], root_ref[...],
                      preferred_element_type=jnp.float32) + b_ref[...]
    o_ref[...] = jnp.maximum(z, 0.0)


def _combine(x, p0, p1, root, b):
    return pl.pallas_call(
        _comb_body,
        grid=(N // BN,),
        in_specs=[
            pl.BlockSpec((BN, D), lambda n: (n, 0)),
            pl.BlockSpec((BN, HID), lambda n: (n, 0)),
            pl.BlockSpec((BN, HID), lambda n: (n, 0)),
            pl.BlockSpec((D, HID), lambda n: (0, 0)),
            pl.BlockSpec((1, HID), lambda n: (0, 0)),
        ],
        out_specs=pl.BlockSpec((BN, HID), lambda n: (n, 0)),
        out_shape=_f32((N, HID)),
    )(x, p0, p1, root, b)


def _pool_body(h_ref, batch_ref, linW_ref, linb_ref, o_ref, acc_sc, cnt_sc):
    i = pl.program_id(0)

    @pl.when(i == 0)
    def _init():
        acc_sc[...] = jnp.zeros_like(acc_sc)
        cnt_sc[...] = jnp.zeros_like(cnt_sc)

    bvec = batch_ref[0, 0, :]
    gids = lax.broadcasted_iota(jnp.int32, (G, BN), 0)
    oh = (gids == bvec[None, :]).astype(jnp.float32)
    acc_sc[...] += jnp.dot(oh, h_ref[...], preferred_element_type=jnp.float32)
    cnt_sc[...] += jnp.broadcast_to(
        jnp.sum(oh, axis=1, keepdims=True), (G, HID))

    @pl.when(i == N // BN - 1)
    def _fin():
        pooled = acc_sc[...] / jnp.maximum(cnt_sc[...], 1.0)
        o_ref[...] = jnp.dot(pooled, linW_ref[...],
                             preferred_element_type=jnp.float32) + linb_ref[...]


def _pool(h, batch3, linW, linb):
    return pl.pallas_call(
        _pool_body,
        grid=(N // BN,),
        in_specs=[
            pl.BlockSpec((BN, HID), lambda n: (n, 0)),
            pl.BlockSpec((1, 1, BN), lambda n: (n, 0, 0)),
            pl.BlockSpec((HID, C), lambda n: (0, 0)),
            pl.BlockSpec((1, C), lambda n: (0, 0)),
        ],
        out_specs=pl.BlockSpec((G, C), lambda n: (0, 0)),
        out_shape=_f32((G, C)),
        scratch_shapes=[
            pltpu.VMEM((G, HID), jnp.float32),
            pltpu.VMEM((G, HID), jnp.float32),
        ],
    )(h, batch3, linW, linb)


def kernel(x, edge_index, edge_type, batch,
           W_rel1, root1, b1, W_rel2, root2, b2, linW, linb):
    src = edge_index[0]
    dst = edge_index[1]
    dst2 = dst.reshape(NW, EPT // B, B)

    w, gidx = _prep(edge_type, src, dst)

    hall1 = _hall(x, W_rel1).reshape(R * N, HID)
    parts1 = _layer_sc(hall1, gidx, dst2, w)
    h1 = _combine(x, parts1[0], parts1[1], root1, b1.reshape(1, HID))

    hall2 = _hall(h1, W_rel2).reshape(R * N, HID)
    parts2 = _layer_sc(hall2, gidx, dst2, w)
    h2 = _combine(h1, parts2[0], parts2[1], root2, b2.reshape(1, HID))

    return _pool(h2, batch.reshape(N // BN, 1, BN), linW,
                 linb.reshape(1, C))


# speedup vs baseline: 28.0981x; 1.4007x over previous
import functools

import jax
import jax.numpy as jnp
from jax import lax
from jax.experimental import pallas as pl
from jax.experimental.pallas import tpu as pltpu
from jax.experimental.pallas import tpu_sc as plsc

N = 10000
E = 320000
D = 128
HID = 128
R = 8
C = 16
G = 64

NC = 2
NS = 16
NW = NC * NS

B = 80
EPT = E // NW
EPS = E // NS
DEGP = 81920
ZCH = DEGP // NS
NPAD = 10240
ROWS_T = NPAD // NS
CH = 2000

_mesh = plsc.VectorSubcoreMesh(core_axis_name="c", subcore_axis_name="s")


def _f32(shape):
    return jax.ShapeDtypeStruct(shape, jnp.float32)


def _prep_body(type_h, src_h, dst_h, w_h, gidx_h,
               deg_sh, deg_v, tb, sb, db, comb_b, ones_b, zb, wb, gb, sem):
    sid = lax.axis_index("s")
    cid = lax.axis_index("c")
    wid = sid * NC + cid

    for i in range(32):
        zb[pl.ds(i * 16, 16)] = jnp.zeros((16,), jnp.float32)
    for i in range(5):
        ones_b[pl.ds(i * 16, 16)] = jnp.ones((16,), jnp.float32)

    def _zero(k, carry):
        pltpu.sync_copy(zb, deg_sh.at[pl.ds(sid * ZCH + k * 512, 512)])
        return carry
    lax.fori_loop(0, ZCH // 512, _zero, 0)
    plsc.subcore_barrier()

    base1 = sid * EPS

    def _count_round(r, carry):
        off = base1 + r * CH
        pltpu.sync_copy(type_h.at[pl.ds(off, CH)], tb)
        pltpu.sync_copy(dst_h.at[pl.ds(off, CH)], db)

        def _blk(b, c2):
            for g in range(5):
                t = tb[pl.ds(b * B + g * 16, 16)]
                dd = db[pl.ds(b * B + g * 16, 16)]
                comb_b[pl.ds(g * 16, 16)] = t * N + dd
            pltpu.sync_copy(ones_b, deg_sh.at[comb_b], add=True)
            return c2
        return lax.fori_loop(0, CH // B, _blk, carry)
    lax.fori_loop(0, EPS // CH, _count_round, 0)
    plsc.subcore_barrier()

    pltpu.sync_copy(deg_sh, deg_v)

    base2 = wid * EPT

    def _emit_round(r, carry):
        off = base2 + r * CH
        pltpu.sync_copy(type_h.at[pl.ds(off, CH)], tb)
        pltpu.sync_copy(src_h.at[pl.ds(off, CH)], sb)
        pltpu.sync_copy(dst_h.at[pl.ds(off, CH)], db)

        def _grp(g, c2):
            t = tb[pl.ds(g * 16, 16)]
            s = sb[pl.ds(g * 16, 16)]
            dd = db[pl.ds(g * 16, 16)]
            degv = plsc.load_gather(deg_v, [t * N + dd])
            wb[pl.ds(g * 16, 16)] = 1.0 / jnp.maximum(degv, 1.0)
            gb[pl.ds(g * 16, 16)] = t * N + s
            return c2
        lax.fori_loop(0, CH // 16, _grp, carry)
        pltpu.sync_copy(wb, w_h.at[pl.ds(off, CH)])
        pltpu.sync_copy(gb, gidx_h.at[pl.ds(off, CH)])
        return carry
    lax.fori_loop(0, EPT // CH, _emit_round, 0)


_prep = pl.kernel(
    _prep_body,
    out_type=(_f32((E,)), jax.ShapeDtypeStruct((E,), jnp.int32)),
    mesh=_mesh,
    compiler_params=pltpu.CompilerParams(needs_layout_passes=False),
    scratch_types=[
        pltpu.VMEM_SHARED((DEGP,), jnp.float32),
        pltpu.VMEM((DEGP,), jnp.float32),
        pltpu.VMEM((CH,), jnp.int32),
        pltpu.VMEM((CH,), jnp.int32),
        pltpu.VMEM((CH,), jnp.int32),
        pltpu.VMEM((B,), jnp.int32),
        pltpu.VMEM((B,), jnp.float32),
        pltpu.VMEM((512,), jnp.float32),
        pltpu.VMEM((CH,), jnp.float32),
        pltpu.VMEM((CH,), jnp.int32),
        pltpu.SemaphoreType.DMA,
    ],
)


RB = 25
NRND = EPT // (RB * B)


def _layer_body(hall_h, gidx_h, dst4_h, w_h, parts_h,
                acc_sh, gi_v, w_r, dst_r, msg0, msg1, sem0, sem1):
    sid = lax.axis_index("s")
    cid = lax.axis_index("c")
    wid = sid * NC + cid

    def _zr(i, carry):
        for c8 in range(8):
            msg0[i, pl.ds(c8 * 16, 16)] = jnp.zeros((16,), jnp.float32)
        return carry
    lax.fori_loop(0, B, _zr, 0)

    def _zero(k, carry):
        pltpu.sync_copy(msg0, acc_sh.at[pl.ds(sid * ROWS_T + k * B, B)])
        return carry
    lax.fori_loop(0, ROWS_T // B, _zero, 0)

    pltpu.sync_copy(gidx_h.at[pl.ds(wid * EPT, EPT)], gi_v)
    plsc.subcore_barrier()

    def _start(b, buf, sem):
        pltpu.async_copy(hall_h.at[gi_v.at[pl.ds(b * B, B)]], buf, sem)

    def _wait(b, buf, sem):
        pltpu.make_async_copy(
            hall_h.at[gi_v.at[pl.ds(b * B, B)]], buf, sem).wait()

    def _proc(bl, buf):
        def _row16(g, c2):
            wv = w_r[pl.ds(bl * B + g * 16, 16)]
            base = g * 16
            for j in range(16):
                wj = wv[j]
                for c8 in range(8):
                    buf[base + j, pl.ds(c8 * 16, 16)] = (
                        buf[base + j, pl.ds(c8 * 16, 16)] * wj)
            return c2
        lax.fori_loop(0, B // 16, _row16, 0)
        pltpu.sync_copy(buf, acc_sh.at[dst_r.at[bl]], add=True)

    def _round(r, carry):
        rb = r * RB
        pltpu.sync_copy(w_h.at[pl.ds(wid * EPT + rb * B, RB * B)], w_r)
        pltpu.sync_copy(dst4_h.at[wid, r], dst_r)
        _start(rb, msg0, sem0)

        def _pair(k, c2):
            b0 = rb + 2 * k
            _start(b0 + 1, msg1, sem1)
            _wait(b0, msg0, sem0)
            _proc(2 * k, msg0)
            _start(b0 + 2, msg0, sem0)
            _wait(b0 + 1, msg1, sem1)
            _proc(2 * k + 1, msg1)
            return c2
        lax.fori_loop(0, (RB - 1) // 2, _pair, carry)
        _wait(rb + RB - 1, msg0, sem0)
        _proc(RB - 1, msg0)
        return carry
    lax.fori_loop(0, NRND, _round, 0)
    plsc.subcore_barrier()

    def _out(k, carry):
        row0 = sid * ROWS_T + k * B
        pltpu.sync_copy(acc_sh.at[pl.ds(row0, B)], msg0)
        pltpu.sync_copy(msg0, parts_h.at[cid, pl.ds(row0, B)])
        return carry
    lax.fori_loop(0, ROWS_T // B, _out, 0)


_layer_sc = pl.kernel(
    _layer_body,
    out_type=_f32((NC, NPAD, HID)),
    mesh=_mesh,
    scratch_types=[
        pltpu.VMEM_SHARED((NPAD, HID), jnp.float32),
        pltpu.VMEM((EPT,), jnp.int32),
        pltpu.VMEM((RB * B,), jnp.float32),
        pltpu.VMEM((RB, B), jnp.int32),
        pltpu.VMEM((B, HID), jnp.float32),
        pltpu.VMEM((B, HID), jnp.float32),
        pltpu.SemaphoreType.DMA,
        pltpu.SemaphoreType.DMA,
    ],
)


BN = 1000


def _mm_body(x_ref, w_ref, o_ref):
    o_ref[0] = jnp.dot(x_ref[...], w_ref[0],
                       preferred_element_type=jnp.float32)


def _hall(x, W_rel):
    return pl.pallas_call(
        _mm_body,
        grid=(R, N // BN),
        in_specs=[
            pl.BlockSpec((BN, D), lambda r, n: (n, 0)),
            pl.BlockSpec((1, D, HID), lambda r, n: (r, 0, 0)),
        ],
        out_specs=pl.BlockSpec((1, BN, HID), lambda r, n: (r, n, 0)),
        out_shape=_f32((R, N, HID)),
    )(x, W_rel)


def _comb_body(x_ref, p0_ref, p1_ref, root_ref, b_ref, o_ref):
    agg = p0_ref[...] + p1_ref[...]
    z = agg + jnp.dot(x_ref[...], root_ref[...],
                      preferred_element_type=jnp.float32) + b_ref[...]
    o_ref[...] = jnp.maximum(z, 0.0)


def _combine(x, p0, p1, root, b):
    return pl.pallas_call(
        _comb_body,
        grid=(N // BN,),
        in_specs=[
            pl.BlockSpec((BN, D), lambda n: (n, 0)),
            pl.BlockSpec((BN, HID), lambda n: (n, 0)),
            pl.BlockSpec((BN, HID), lambda n: (n, 0)),
            pl.BlockSpec((D, HID), lambda n: (0, 0)),
            pl.BlockSpec((1, HID), lambda n: (0, 0)),
        ],
        out_specs=pl.BlockSpec((BN, HID), lambda n: (n, 0)),
        out_shape=_f32((N, HID)),
    )(x, p0, p1, root, b)


def _pool_body(h_ref, batch_ref, linW_ref, linb_ref, o_ref, acc_sc, cnt_sc):
    i = pl.program_id(0)

    @pl.when(i == 0)
    def _init():
        acc_sc[...] = jnp.zeros_like(acc_sc)
        cnt_sc[...] = jnp.zeros_like(cnt_sc)

    bvec = batch_ref[0, 0, :]
    gids = lax.broadcasted_iota(jnp.int32, (G, BN), 0)
    oh = (gids == bvec[None, :]).astype(jnp.float32)
    acc_sc[...] += jnp.dot(oh, h_ref[...], preferred_element_type=jnp.float32)
    cnt_sc[...] += jnp.broadcast_to(
        jnp.sum(oh, axis=1, keepdims=True), (G, HID))

    @pl.when(i == N // BN - 1)
    def _fin():
        pooled = acc_sc[...] / jnp.maximum(cnt_sc[...], 1.0)
        o_ref[...] = jnp.dot(pooled, linW_ref[...],
                             preferred_element_type=jnp.float32) + linb_ref[...]


def _pool(h, batch3, linW, linb):
    return pl.pallas_call(
        _pool_body,
        grid=(N // BN,),
        in_specs=[
            pl.BlockSpec((BN, HID), lambda n: (n, 0)),
            pl.BlockSpec((1, 1, BN), lambda n: (n, 0, 0)),
            pl.BlockSpec((HID, C), lambda n: (0, 0)),
            pl.BlockSpec((1, C), lambda n: (0, 0)),
        ],
        out_specs=pl.BlockSpec((G, C), lambda n: (0, 0)),
        out_shape=_f32((G, C)),
        scratch_shapes=[
            pltpu.VMEM((G, HID), jnp.float32),
            pltpu.VMEM((G, HID), jnp.float32),
        ],
    )(h, batch3, linW, linb)


def kernel(x, edge_index, edge_type, batch,
           W_rel1, root1, b1, W_rel2, root2, b2, linW, linb):
    src = edge_index[0]
    dst = edge_index[1]
    dst2 = dst.reshape(NW, NRND, RB, B)

    w, gidx = _prep(edge_type, src, dst)

    hall1 = _hall(x, W_rel1).reshape(R * N, HID)
    parts1 = _layer_sc(hall1, gidx, dst2, w)
    h1 = _combine(x, parts1[0], parts1[1], root1, b1.reshape(1, HID))

    hall2 = _hall(h1, W_rel2).reshape(R * N, HID)
    parts2 = _layer_sc(hall2, gidx, dst2, w)
    h2 = _combine(h1, parts2[0], parts2[1], root2, b2.reshape(1, HID))

    return _pool(h2, batch.reshape(N // BN, 1, BN), linW,
                 linb.reshape(1, C))


# speedup vs baseline: 28.9214x; 1.0293x over previous
import functools

import jax
import jax.numpy as jnp
from jax import lax
from jax.experimental import pallas as pl
from jax.experimental.pallas import tpu as pltpu
from jax.experimental.pallas import tpu_sc as plsc

N = 10000
E = 320000
D = 128
HID = 128
R = 8
C = 16
G = 64

NC = 2
NS = 16
NW = NC * NS

B = 80
EPT = E // NW
EPS = E // NS
DEGP = 81920
ZCH = DEGP // NS
NPAD = 10240
ROWS_T = NPAD // NS
CH = 2000

_mesh = plsc.VectorSubcoreMesh(core_axis_name="c", subcore_axis_name="s")


def _f32(shape):
    return jax.ShapeDtypeStruct(shape, jnp.float32)


def _prep_body(type_h, src_h, dst_h, w_h, gidx_h,
               deg_sh, deg_v, tb, sb, db, comb_b, ones_b, zb, wb, gb, sem):
    sid = lax.axis_index("s")
    cid = lax.axis_index("c")
    wid = sid * NC + cid

    for i in range(32):
        zb[pl.ds(i * 16, 16)] = jnp.zeros((16,), jnp.float32)
    for i in range(5):
        ones_b[pl.ds(i * 16, 16)] = jnp.ones((16,), jnp.float32)

    def _zero(k, carry):
        pltpu.sync_copy(zb, deg_sh.at[pl.ds(sid * ZCH + k * 512, 512)])
        return carry
    lax.fori_loop(0, ZCH // 512, _zero, 0)
    plsc.subcore_barrier()

    base1 = sid * EPS

    def _count_round(r, carry):
        off = base1 + r * CH
        pltpu.sync_copy(type_h.at[pl.ds(off, CH)], tb)
        pltpu.sync_copy(dst_h.at[pl.ds(off, CH)], db)

        def _blk(b, c2):
            for g in range(5):
                t = tb[pl.ds(b * B + g * 16, 16)]
                dd = db[pl.ds(b * B + g * 16, 16)]
                comb_b[pl.ds(g * 16, 16)] = t * N + dd
            pltpu.sync_copy(ones_b, deg_sh.at[comb_b], add=True)
            return c2
        return lax.fori_loop(0, CH // B, _blk, carry)
    lax.fori_loop(0, EPS // CH, _count_round, 0)
    plsc.subcore_barrier()

    pltpu.sync_copy(deg_sh, deg_v)

    base2 = wid * EPT

    def _emit_round(r, carry):
        off = base2 + r * CH
        pltpu.sync_copy(type_h.at[pl.ds(off, CH)], tb)
        pltpu.sync_copy(src_h.at[pl.ds(off, CH)], sb)
        pltpu.sync_copy(dst_h.at[pl.ds(off, CH)], db)

        def _grp(g, c2):
            t = tb[pl.ds(g * 16, 16)]
            s = sb[pl.ds(g * 16, 16)]
            dd = db[pl.ds(g * 16, 16)]
            degv = plsc.load_gather(deg_v, [t * N + dd])
            wb[pl.ds(g * 16, 16)] = 1.0 / jnp.maximum(degv, 1.0)
            gb[pl.ds(g * 16, 16)] = t * N + s
            return c2
        lax.fori_loop(0, CH // 16, _grp, carry)
        pltpu.sync_copy(wb, w_h.at[pl.ds(off, CH)])
        pltpu.sync_copy(gb, gidx_h.at[pl.ds(off, CH)])
        return carry
    lax.fori_loop(0, EPT // CH, _emit_round, 0)


_prep = pl.kernel(
    _prep_body,
    out_type=(_f32((E,)), jax.ShapeDtypeStruct((E,), jnp.int32)),
    mesh=_mesh,
    compiler_params=pltpu.CompilerParams(needs_layout_passes=False),
    scratch_types=[
        pltpu.VMEM_SHARED((DEGP,), jnp.float32),
        pltpu.VMEM((DEGP,), jnp.float32),
        pltpu.VMEM((CH,), jnp.int32),
        pltpu.VMEM((CH,), jnp.int32),
        pltpu.VMEM((CH,), jnp.int32),
        pltpu.VMEM((B,), jnp.int32),
        pltpu.VMEM((B,), jnp.float32),
        pltpu.VMEM((512,), jnp.float32),
        pltpu.VMEM((CH,), jnp.float32),
        pltpu.VMEM((CH,), jnp.int32),
        pltpu.SemaphoreType.DMA,
    ],
)


RB = 25
NRND = EPT // (RB * B)


def _layer_body(hall_h, gidx_h, dst4_h, w_h, parts_h,
                acc_sh, gi_r, w_r, dst_r, msg0, msg1, msg2,
                gs0, gs1, gs2, ss0, ss1, ss2):
    sid = lax.axis_index("s")
    cid = lax.axis_index("c")
    wid = sid * NC + cid

    bufs = (msg0, msg1, msg2)
    gsems = (gs0, gs1, gs2)
    ssems = (ss0, ss1, ss2)

    def _zr(i, carry):
        for c8 in range(8):
            msg0[i, pl.ds(c8 * 16, 16)] = jnp.zeros((16,), jnp.float32)
        return carry
    lax.fori_loop(0, B, _zr, 0)

    def _zero(k, carry):
        pltpu.sync_copy(msg0, acc_sh.at[pl.ds(sid * ROWS_T + k * B, B)])
        return carry
    lax.fori_loop(0, ROWS_T // B, _zero, 0)
    plsc.subcore_barrier()

    def _round(r, carry):
        base_e = wid * EPT + r * (RB * B)
        pltpu.sync_copy(gidx_h.at[pl.ds(base_e, RB * B)], gi_r)
        pltpu.sync_copy(w_h.at[pl.ds(base_e, RB * B)], w_r)
        pltpu.sync_copy(dst4_h.at[wid, r], dst_r)

        def _g(m, j):
            pltpu.async_copy(hall_h.at[gi_r.at[pl.ds(m * B, B)]],
                             bufs[j], gsems[j])

        def _gwait(m, j):
            pltpu.make_async_copy(hall_h.at[gi_r.at[pl.ds(m * B, B)]],
                                  bufs[j], gsems[j]).wait()

        def _s(m, j):
            pltpu.async_copy(bufs[j], acc_sh.at[dst_r.at[m]], ssems[j],
                             add=True)

        def _swait(m, j):
            pltpu.make_async_copy(bufs[j], acc_sh.at[dst_r.at[m]],
                                  ssems[j]).wait()

        def _scale(m, buf):
            def _row16(g2, c2):
                wv = w_r[pl.ds(m * B + g2 * 16, 16)]
                base = g2 * 16
                for j in range(16):
                    wj = wv[j]
                    for c8 in range(8):
                        buf[base + j, pl.ds(c8 * 16, 16)] = (
                            buf[base + j, pl.ds(c8 * 16, 16)] * wj)
                return c2
            lax.fori_loop(0, B // 16, _row16, 0)

        _g(0, 0)
        _g(1, 1)
        _g(2, 2)
        _gwait(0, 0); _scale(0, msg0); _s(0, 0)
        _swait(0, 0); _g(3, 0)
        _gwait(1, 1); _scale(1, msg1); _s(1, 1)
        _swait(1, 1); _g(4, 1)
        _gwait(2, 2); _scale(2, msg2); _s(2, 2)

        def _grp(g, c2):
            m0 = 3 + 3 * g
            for i in range(3):
                m = m0 + i
                jn = (i + 2) % 3
                _swait(m - 1, jn)
                _g(m + 2, jn)
                _gwait(m, i)
                _scale(m, bufs[i])
                _s(m, i)
            return c2
        lax.fori_loop(0, 6, _grp, 0)

        _swait(20, 2); _g(23, 2)
        _gwait(21, 0); _scale(21, msg0); _s(21, 0)
        _swait(21, 0); _g(24, 0)
        _gwait(22, 1); _scale(22, msg1); _s(22, 1)
        _gwait(23, 2); _scale(23, msg2); _s(23, 2)
        _gwait(24, 0); _scale(24, msg0); _s(24, 0)
        _swait(22, 1)
        _swait(23, 2)
        _swait(24, 0)
        return carry
    lax.fori_loop(0, NRND, _round, 0)
    plsc.subcore_barrier()

    def _out(k, carry):
        row0 = sid * ROWS_T + k * B
        pltpu.sync_copy(acc_sh.at[pl.ds(row0, B)], msg0)
        pltpu.sync_copy(msg0, parts_h.at[cid, pl.ds(row0, B)])
        return carry
    lax.fori_loop(0, ROWS_T // B, _out, 0)


_layer_sc = pl.kernel(
    _layer_body,
    out_type=_f32((NC, NPAD, HID)),
    mesh=_mesh,
    scratch_types=[
        pltpu.VMEM_SHARED((NPAD, HID), jnp.float32),
        pltpu.VMEM((RB * B,), jnp.int32),
        pltpu.VMEM((RB * B,), jnp.float32),
        pltpu.VMEM((RB, B), jnp.int32),
        pltpu.VMEM((B, HID), jnp.float32),
        pltpu.VMEM((B, HID), jnp.float32),
        pltpu.VMEM((B, HID), jnp.float32),
        pltpu.SemaphoreType.DMA,
        pltpu.SemaphoreType.DMA,
        pltpu.SemaphoreType.DMA,
        pltpu.SemaphoreType.DMA,
        pltpu.SemaphoreType.DMA,
        pltpu.SemaphoreType.DMA,
    ],
)


BN = 1000


def _mm_body(x_ref, w_ref, o_ref):
    o_ref[0] = jnp.dot(x_ref[...], w_ref[0],
                       preferred_element_type=jnp.float32)


def _hall(x, W_rel):
    return pl.pallas_call(
        _mm_body,
        grid=(R, N // BN),
        in_specs=[
            pl.BlockSpec((BN, D), lambda r, n: (n, 0)),
            pl.BlockSpec((1, D, HID), lambda r, n: (r, 0, 0)),
        ],
        out_specs=pl.BlockSpec((1, BN, HID), lambda r, n: (r, n, 0)),
        out_shape=_f32((R, N, HID)),
    )(x, W_rel)


def _comb_body(x_ref, p0_ref, p1_ref, root_ref, b_ref, o_ref):
    agg = p0_ref[...] + p1_ref[...]
    z = agg + jnp.dot(x_ref[...], root_ref[...],
                      preferred_element_type=jnp.float32) + b_ref[...]
    o_ref[...] = jnp.maximum(z, 0.0)


def _combine(x, p0, p1, root, b):
    return pl.pallas_call(
        _comb_body,
        grid=(N // BN,),
        in_specs=[
            pl.BlockSpec((BN, D), lambda n: (n, 0)),
            pl.BlockSpec((BN, HID), lambda n: (n, 0)),
            pl.BlockSpec((BN, HID), lambda n: (n, 0)),
            pl.BlockSpec((D, HID), lambda n: (0, 0)),
            pl.BlockSpec((1, HID), lambda n: (0, 0)),
        ],
        out_specs=pl.BlockSpec((BN, HID), lambda n: (n, 0)),
        out_shape=_f32((N, HID)),
    )(x, p0, p1, root, b)


def _pool_body(h_ref, batch_ref, linW_ref, linb_ref, o_ref, acc_sc, cnt_sc):
    i = pl.program_id(0)

    @pl.when(i == 0)
    def _init():
        acc_sc[...] = jnp.zeros_like(acc_sc)
        cnt_sc[...] = jnp.zeros_like(cnt_sc)

    bvec = batch_ref[0, 0, :]
    gids = lax.broadcasted_iota(jnp.int32, (G, BN), 0)
    oh = (gids == bvec[None, :]).astype(jnp.float32)
    acc_sc[...] += jnp.dot(oh, h_ref[...], preferred_element_type=jnp.float32)
    cnt_sc[...] += jnp.broadcast_to(
        jnp.sum(oh, axis=1, keepdims=True), (G, HID))

    @pl.when(i == N // BN - 1)
    def _fin():
        pooled = acc_sc[...] / jnp.maximum(cnt_sc[...], 1.0)
        o_ref[...] = jnp.dot(pooled, linW_ref[...],
                             preferred_element_type=jnp.float32) + linb_ref[...]


def _pool(h, batch3, linW, linb):
    return pl.pallas_call(
        _pool_body,
        grid=(N // BN,),
        in_specs=[
            pl.BlockSpec((BN, HID), lambda n: (n, 0)),
            pl.BlockSpec((1, 1, BN), lambda n: (n, 0, 0)),
            pl.BlockSpec((HID, C), lambda n: (0, 0)),
            pl.BlockSpec((1, C), lambda n: (0, 0)),
        ],
        out_specs=pl.BlockSpec((G, C), lambda n: (0, 0)),
        out_shape=_f32((G, C)),
        scratch_shapes=[
            pltpu.VMEM((G, HID), jnp.float32),
            pltpu.VMEM((G, HID), jnp.float32),
        ],
    )(h, batch3, linW, linb)


def kernel(x, edge_index, edge_type, batch,
           W_rel1, root1, b1, W_rel2, root2, b2, linW, linb):
    src = edge_index[0]
    dst = edge_index[1]
    dst2 = dst.reshape(NW, NRND, RB, B)

    w, gidx = _prep(edge_type, src, dst)

    hall1 = _hall(x, W_rel1).reshape(R * N, HID)
    parts1 = _layer_sc(hall1, gidx, dst2, w)
    h1 = _combine(x, parts1[0], parts1[1], root1, b1.reshape(1, HID))

    hall2 = _hall(h1, W_rel2).reshape(R * N, HID)
    parts2 = _layer_sc(hall2, gidx, dst2, w)
    h2 = _combine(h1, parts2[0], parts2[1], root2, b2.reshape(1, HID))

    return _pool(h2, batch.reshape(N // BN, 1, BN), linW,
                 linb.reshape(1, C))


# speedup vs baseline: 32.9815x; 1.1404x over previous
import functools

import jax
import jax.numpy as jnp
from jax import lax
from jax.experimental import pallas as pl
from jax.experimental.pallas import tpu as pltpu
from jax.experimental.pallas import tpu_sc as plsc

N = 10000
E = 320000
D = 128
HID = 128
R = 8
C = 16
G = 64

NC = 2
NS = 16
NW = NC * NS

B = 80
EPT = E // NW
EPS = E // NS
DEGP = 81920
ZCH = DEGP // NS
NPAD = 10240
ROWS_T = NPAD // NS
CH = 2000

_mesh = plsc.VectorSubcoreMesh(core_axis_name="c", subcore_axis_name="s")


def _f32(shape):
    return jax.ShapeDtypeStruct(shape, jnp.float32)


def _prep_body(type_h, src_h, dst_h, w_h, gidx_h,
               deg_sh, deg_v, tb, sb, db, comb_b, ones_b, zb, wb, gb, sem):
    sid = lax.axis_index("s")
    cid = lax.axis_index("c")
    wid = sid * NC + cid

    for i in range(32):
        zb[pl.ds(i * 16, 16)] = jnp.zeros((16,), jnp.float32)
    for i in range(5):
        ones_b[pl.ds(i * 16, 16)] = jnp.ones((16,), jnp.float32)

    def _zero(k, carry):
        pltpu.sync_copy(zb, deg_sh.at[pl.ds(sid * ZCH + k * 512, 512)])
        return carry
    lax.fori_loop(0, ZCH // 512, _zero, 0)
    plsc.subcore_barrier()

    base1 = sid * EPS

    def _count_round(r, carry):
        off = base1 + r * CH
        pltpu.sync_copy(type_h.at[pl.ds(off, CH)], tb)
        pltpu.sync_copy(dst_h.at[pl.ds(off, CH)], db)

        def _blk(b, c2):
            for g in range(5):
                t = tb[pl.ds(b * B + g * 16, 16)]
                dd = db[pl.ds(b * B + g * 16, 16)]
                comb_b[pl.ds(g * 16, 16)] = t * N + dd
            pltpu.sync_copy(ones_b, deg_sh.at[comb_b], add=True)
            return c2
        return lax.fori_loop(0, CH // B, _blk, carry)
    lax.fori_loop(0, EPS // CH, _count_round, 0)
    plsc.subcore_barrier()

    pltpu.sync_copy(deg_sh, deg_v)

    base2 = wid * EPT

    def _emit_round(r, carry):
        off = base2 + r * CH
        pltpu.sync_copy(type_h.at[pl.ds(off, CH)], tb)
        pltpu.sync_copy(src_h.at[pl.ds(off, CH)], sb)
        pltpu.sync_copy(dst_h.at[pl.ds(off, CH)], db)

        def _grp(g, c2):
            t = tb[pl.ds(g * 16, 16)]
            s = sb[pl.ds(g * 16, 16)]
            dd = db[pl.ds(g * 16, 16)]
            degv = plsc.load_gather(deg_v, [t * N + dd])
            wb[pl.ds(g * 16, 16)] = 1.0 / jnp.maximum(degv, 1.0)
            gb[pl.ds(g * 16, 16)] = t * N + s
            return c2
        lax.fori_loop(0, CH // 16, _grp, carry)
        pltpu.sync_copy(wb, w_h.at[pl.ds(off, CH)])
        pltpu.sync_copy(gb, gidx_h.at[pl.ds(off, CH)])
        return carry
    lax.fori_loop(0, EPT // CH, _emit_round, 0)


_prep = pl.kernel(
    _prep_body,
    out_type=(_f32((E,)), jax.ShapeDtypeStruct((E,), jnp.int32)),
    mesh=_mesh,
    compiler_params=pltpu.CompilerParams(needs_layout_passes=False),
    scratch_types=[
        pltpu.VMEM_SHARED((DEGP,), jnp.float32),
        pltpu.VMEM((DEGP,), jnp.float32),
        pltpu.VMEM((CH,), jnp.int32),
        pltpu.VMEM((CH,), jnp.int32),
        pltpu.VMEM((CH,), jnp.int32),
        pltpu.VMEM((B,), jnp.int32),
        pltpu.VMEM((B,), jnp.float32),
        pltpu.VMEM((512,), jnp.float32),
        pltpu.VMEM((CH,), jnp.float32),
        pltpu.VMEM((CH,), jnp.int32),
        pltpu.SemaphoreType.DMA,
    ],
)


RB = 25
NRND = EPT // (RB * B)


def _layer_body(hall_h, gidx_h, dst4_h, w_h, parts_h,
                acc_sh, gi_r, w_r, dst_r, msg0, msg1, msg2,
                gs0, gs1, gs2, ss0, ss1, ss2):
    sid = lax.axis_index("s")
    cid = lax.axis_index("c")
    wid = sid * NC + cid

    bufs = (msg0, msg1, msg2)
    gsems = (gs0, gs1, gs2)
    ssems = (ss0, ss1, ss2)

    def _zr(i, carry):
        for c8 in range(8):
            msg0[i, pl.ds(c8 * 16, 16)] = jnp.zeros((16,), jnp.float32)
        return carry
    lax.fori_loop(0, B, _zr, 0)

    def _zero(k, carry):
        pltpu.sync_copy(msg0, acc_sh.at[pl.ds(sid * ROWS_T + k * B, B)])
        return carry
    lax.fori_loop(0, ROWS_T // B, _zero, 0)
    plsc.subcore_barrier()

    def _round(r, carry):
        base_e = wid * EPT + r * (RB * B)
        pltpu.sync_copy(gidx_h.at[pl.ds(base_e, RB * B)], gi_r)
        pltpu.sync_copy(w_h.at[pl.ds(base_e, RB * B)], w_r)
        pltpu.sync_copy(dst4_h.at[wid, r], dst_r)

        def _g(m, j):
            pltpu.async_copy(hall_h.at[gi_r.at[pl.ds(m * B, B)]],
                             bufs[j], gsems[j])

        def _gwait(m, j):
            pltpu.make_async_copy(hall_h.at[gi_r.at[pl.ds(m * B, B)]],
                                  bufs[j], gsems[j]).wait()

        def _s(m, j):
            pltpu.async_copy(bufs[j], acc_sh.at[dst_r.at[m]], ssems[j],
                             add=True)

        def _swait(m, j):
            pltpu.make_async_copy(bufs[j], acc_sh.at[dst_r.at[m]],
                                  ssems[j]).wait()

        def _scale(m, buf):
            def _row16(g2, c2):
                wv = w_r[pl.ds(m * B + g2 * 16, 16)]
                base = g2 * 16
                for j in range(16):
                    wj = wv[j]
                    for c8 in range(8):
                        buf[base + j, pl.ds(c8 * 16, 16)] = (
                            buf[base + j, pl.ds(c8 * 16, 16)] * wj)
                return c2
            lax.fori_loop(0, B // 16, _row16, 0)

        _g(0, 0)
        _g(1, 1)
        _g(2, 2)
        _gwait(0, 0); _scale(0, msg0); _s(0, 0)
        _swait(0, 0); _g(3, 0)
        _gwait(1, 1); _scale(1, msg1); _s(1, 1)
        _swait(1, 1); _g(4, 1)
        _gwait(2, 2); _scale(2, msg2); _s(2, 2)

        def _grp(g, c2):
            m0 = 3 + 3 * g
            for i in range(3):
                m = m0 + i
                jn = (i + 2) % 3
                _swait(m - 1, jn)
                _g(m + 2, jn)
                _gwait(m, i)
                _scale(m, bufs[i])
                _s(m, i)
            return c2
        lax.fori_loop(0, 6, _grp, 0)

        _swait(20, 2); _g(23, 2)
        _gwait(21, 0); _scale(21, msg0); _s(21, 0)
        _swait(21, 0); _g(24, 0)
        _gwait(22, 1); _scale(22, msg1); _s(22, 1)
        _gwait(23, 2); _scale(23, msg2); _s(23, 2)
        _gwait(24, 0); _scale(24, msg0); _s(24, 0)
        _swait(22, 1)
        _swait(23, 2)
        _swait(24, 0)
        return carry
    lax.fori_loop(0, NRND, _round, 0)
    plsc.subcore_barrier()

    def _out(k, carry):
        row0 = sid * ROWS_T + k * B
        pltpu.sync_copy(acc_sh.at[pl.ds(row0, B)], msg0)
        pltpu.sync_copy(msg0, parts_h.at[cid, pl.ds(row0, B)])
        return carry
    lax.fori_loop(0, ROWS_T // B, _out, 0)


_layer_sc = pl.kernel(
    _layer_body,
    out_type=_f32((NC, NPAD, HID)),
    mesh=_mesh,
    scratch_types=[
        pltpu.VMEM_SHARED((NPAD, HID), jnp.float32),
        pltpu.VMEM((RB * B,), jnp.int32),
        pltpu.VMEM((RB * B,), jnp.float32),
        pltpu.VMEM((RB, B), jnp.int32),
        pltpu.VMEM((B, HID), jnp.float32),
        pltpu.VMEM((B, HID), jnp.float32),
        pltpu.VMEM((B, HID), jnp.float32),
        pltpu.SemaphoreType.DMA,
        pltpu.SemaphoreType.DMA,
        pltpu.SemaphoreType.DMA,
        pltpu.SemaphoreType.DMA,
        pltpu.SemaphoreType.DMA,
        pltpu.SemaphoreType.DMA,
    ],
)


BN = 1000


def _mm_body(x_ref, w_ref, o_ref):
    o_ref[0] = jnp.dot(x_ref[...], w_ref[0],
                       preferred_element_type=jnp.float32)


def _hall(x, W_rel):
    return pl.pallas_call(
        _mm_body,
        grid=(R, N // BN),
        in_specs=[
            pl.BlockSpec((BN, D), lambda r, n: (n, 0)),
            pl.BlockSpec((1, D, HID), lambda r, n: (r, 0, 0)),
        ],
        out_specs=pl.BlockSpec((1, BN, HID), lambda r, n: (r, n, 0)),
        out_shape=_f32((R, N, HID)),
    )(x, W_rel)


def _comb_hall_body(x_ref, p0_ref, p1_ref, root_ref, b_ref, w2_ref,
                    h_ref, hall_ref):
    agg = p0_ref[...] + p1_ref[...]
    z = agg + jnp.dot(x_ref[...], root_ref[...],
                      preferred_element_type=jnp.float32) + b_ref[...]
    h = jnp.maximum(z, 0.0)
    h_ref[...] = h
    for r in range(R):
        hall_ref[r] = jnp.dot(h, w2_ref[r],
                              preferred_element_type=jnp.float32)


def _comb_hall(x, p0, p1, root, b, W_rel2):
    return pl.pallas_call(
        _comb_hall_body,
        grid=(N // BN,),
        in_specs=[
            pl.BlockSpec((BN, D), lambda n: (n, 0)),
            pl.BlockSpec((BN, HID), lambda n: (n, 0)),
            pl.BlockSpec((BN, HID), lambda n: (n, 0)),
            pl.BlockSpec((D, HID), lambda n: (0, 0)),
            pl.BlockSpec((1, HID), lambda n: (0, 0)),
            pl.BlockSpec((R, HID, HID), lambda n: (0, 0, 0)),
        ],
        out_specs=[
            pl.BlockSpec((BN, HID), lambda n: (n, 0)),
            pl.BlockSpec((R, BN, HID), lambda n: (0, n, 0)),
        ],
        out_shape=[_f32((N, HID)), _f32((R, N, HID))],
    )(x, p0, p1, root, b, W_rel2)


def _comb_pool_body(h1_ref, p0_ref, p1_ref, root_ref, b_ref, batch_ref,
                    linW_ref, linb_ref, o_ref, acc_sc, cnt_sc):
    i = pl.program_id(0)

    @pl.when(i == 0)
    def _init():
        acc_sc[...] = jnp.zeros_like(acc_sc)
        cnt_sc[...] = jnp.zeros_like(cnt_sc)

    agg = p0_ref[...] + p1_ref[...]
    z = agg + jnp.dot(h1_ref[...], root_ref[...],
                      preferred_element_type=jnp.float32) + b_ref[...]
    h2 = jnp.maximum(z, 0.0)

    bvec = batch_ref[0, 0, :]
    gids = lax.broadcasted_iota(jnp.int32, (G, BN), 0)
    oh = (gids == bvec[None, :]).astype(jnp.float32)
    acc_sc[...] += jnp.dot(oh, h2, preferred_element_type=jnp.float32)
    cnt_sc[...] += jnp.broadcast_to(
        jnp.sum(oh, axis=1, keepdims=True), (G, HID))

    @pl.when(i == N // BN - 1)
    def _fin():
        pooled = acc_sc[...] / jnp.maximum(cnt_sc[...], 1.0)
        o_ref[...] = jnp.dot(pooled, linW_ref[...],
                             preferred_element_type=jnp.float32) + linb_ref[...]


def _comb_pool(h1, p0, p1, root, b, batch3, linW, linb):
    return pl.pallas_call(
        _comb_pool_body,
        grid=(N // BN,),
        in_specs=[
            pl.BlockSpec((BN, HID), lambda n: (n, 0)),
            pl.BlockSpec((BN, HID), lambda n: (n, 0)),
            pl.BlockSpec((BN, HID), lambda n: (n, 0)),
            pl.BlockSpec((HID, HID), lambda n: (0, 0)),
            pl.BlockSpec((1, HID), lambda n: (0, 0)),
            pl.BlockSpec((1, 1, BN), lambda n: (n, 0, 0)),
            pl.BlockSpec((HID, C), lambda n: (0, 0)),
            pl.BlockSpec((1, C), lambda n: (0, 0)),
        ],
        out_specs=pl.BlockSpec((G, C), lambda n: (0, 0)),
        out_shape=_f32((G, C)),
        scratch_shapes=[
            pltpu.VMEM((G, HID), jnp.float32),
            pltpu.VMEM((G, HID), jnp.float32),
        ],
    )(h1, p0, p1, root, b, batch3, linW, linb)


def kernel(x, edge_index, edge_type, batch,
           W_rel1, root1, b1, W_rel2, root2, b2, linW, linb):
    src = edge_index[0]
    dst = edge_index[1]
    dst2 = dst.reshape(NW, NRND, RB, B)

    w, gidx = _prep(edge_type, src, dst)

    hall1 = _hall(x, W_rel1).reshape(R * N, HID)
    parts1 = _layer_sc(hall1, gidx, dst2, w)
    h1, hall2 = _comb_hall(x, parts1[0], parts1[1], root1,
                           b1.reshape(1, HID), W_rel2)

    parts2 = _layer_sc(hall2.reshape(R * N, HID), gidx, dst2, w)
    return _comb_pool(h1, parts2[0], parts2[1], root2, b2.reshape(1, HID),
                      batch.reshape(N // BN, 1, BN), linW, linb.reshape(1, C))


# speedup vs baseline: 33.4350x; 1.0137x over previous
import functools

import jax
import jax.numpy as jnp
from jax import lax
from jax.experimental import pallas as pl
from jax.experimental.pallas import tpu as pltpu
from jax.experimental.pallas import tpu_sc as plsc

N = 10000
E = 320000
D = 128
HID = 128
R = 8
C = 16
G = 64

NC = 2
NS = 16
NW = NC * NS

B = 80
EPT = E // NW
EPS = E // NS
DEGP = 81920
ZCH = DEGP // NS
NPAD = 10240
ROWS_T = NPAD // NS
CH = 2000

_mesh = plsc.VectorSubcoreMesh(core_axis_name="c", subcore_axis_name="s")


def _f32(shape):
    return jax.ShapeDtypeStruct(shape, jnp.float32)


def _prep_body(type_h, src_h, dst_h, w_h, gidx_h,
               deg_sh, deg_v, tb, sb, db, comb_b, comb2_b, ones_b, zb,
               wb, gb, sem, sem2):
    sid = lax.axis_index("s")
    cid = lax.axis_index("c")
    wid = sid * NC + cid

    for i in range(32):
        zb[pl.ds(i * 16, 16)] = jnp.zeros((16,), jnp.float32)
    for i in range(5):
        ones_b[pl.ds(i * 16, 16)] = jnp.ones((16,), jnp.float32)

    def _zero(k, carry):
        pltpu.sync_copy(zb, deg_sh.at[pl.ds(sid * ZCH + k * 512, 512)])
        return carry
    lax.fori_loop(0, ZCH // 512, _zero, 0)
    plsc.subcore_barrier()

    base1 = sid * EPS

    def _build(blk, cbuf):
        for g in range(5):
            t = tb[pl.ds(blk * B + g * 16, 16)]
            dd = db[pl.ds(blk * B + g * 16, 16)]
            cbuf[pl.ds(g * 16, 16)] = t * N + dd

    def _count_round(r, carry):
        off = base1 + r * CH
        pltpu.sync_copy(type_h.at[pl.ds(off, CH)], tb)
        pltpu.sync_copy(dst_h.at[pl.ds(off, CH)], db)

        _build(0, comb_b)
        pltpu.async_copy(ones_b, deg_sh.at[comb_b], sem, add=True)

        def _blk(k, c2):
            _build(2 * k + 1, comb2_b)
            pltpu.async_copy(ones_b, deg_sh.at[comb2_b], sem2, add=True)
            pltpu.make_async_copy(ones_b, deg_sh.at[comb_b], sem).wait()
            _build(2 * k + 2, comb_b)
            pltpu.async_copy(ones_b, deg_sh.at[comb_b], sem, add=True)
            pltpu.make_async_copy(ones_b, deg_sh.at[comb2_b], sem2).wait()
            return c2
        lax.fori_loop(0, (CH // B) // 2, _blk, carry)
        pltpu.make_async_copy(ones_b, deg_sh.at[comb_b], sem).wait()
        return carry
    lax.fori_loop(0, EPS // CH, _count_round, 0)
    plsc.subcore_barrier()

    pltpu.sync_copy(deg_sh, deg_v)

    base2 = wid * EPT

    def _emit_round(r, carry):
        off = base2 + r * CH
        pltpu.sync_copy(type_h.at[pl.ds(off, CH)], tb)
        pltpu.sync_copy(src_h.at[pl.ds(off, CH)], sb)
        pltpu.sync_copy(dst_h.at[pl.ds(off, CH)], db)

        def _grp(g, c2):
            t = tb[pl.ds(g * 16, 16)]
            s = sb[pl.ds(g * 16, 16)]
            dd = db[pl.ds(g * 16, 16)]
            degv = plsc.load_gather(deg_v, [t * N + dd])
            wb[pl.ds(g * 16, 16)] = 1.0 / jnp.maximum(degv, 1.0)
            gb[pl.ds(g * 16, 16)] = t * N + s
            return c2
        lax.fori_loop(0, CH // 16, _grp, carry)
        pltpu.sync_copy(wb, w_h.at[pl.ds(off, CH)])
        pltpu.sync_copy(gb, gidx_h.at[pl.ds(off, CH)])
        return carry
    lax.fori_loop(0, EPT // CH, _emit_round, 0)


_prep = pl.kernel(
    _prep_body,
    out_type=(_f32((E,)), jax.ShapeDtypeStruct((E,), jnp.int32)),
    mesh=_mesh,
    compiler_params=pltpu.CompilerParams(needs_layout_passes=False),
    scratch_types=[
        pltpu.VMEM_SHARED((DEGP,), jnp.float32),
        pltpu.VMEM((DEGP,), jnp.float32),
        pltpu.VMEM((CH,), jnp.int32),
        pltpu.VMEM((CH,), jnp.int32),
        pltpu.VMEM((CH,), jnp.int32),
        pltpu.VMEM((B,), jnp.int32),
        pltpu.VMEM((B,), jnp.int32),
        pltpu.VMEM((B,), jnp.float32),
        pltpu.VMEM((512,), jnp.float32),
        pltpu.VMEM((CH,), jnp.float32),
        pltpu.VMEM((CH,), jnp.int32),
        pltpu.SemaphoreType.DMA,
        pltpu.SemaphoreType.DMA,
    ],
)


RB = 25
NRND = EPT // (RB * B)


def _layer_body(hall_h, gidx_h, dst4_h, w_h, parts_h,
                acc_sh, gi_r, w_r, dst_r, msg0, msg1, msg2,
                gs0, gs1, gs2, ss0, ss1, ss2):
    sid = lax.axis_index("s")
    cid = lax.axis_index("c")
    wid = sid * NC + cid

    bufs = (msg0, msg1, msg2)
    gsems = (gs0, gs1, gs2)
    ssems = (ss0, ss1, ss2)

    def _zr(i, carry):
        for c8 in range(8):
            msg0[i, pl.ds(c8 * 16, 16)] = jnp.zeros((16,), jnp.float32)
        return carry
    lax.fori_loop(0, B, _zr, 0)

    def _zero(k, carry):
        pltpu.sync_copy(msg0, acc_sh.at[pl.ds(sid * ROWS_T + k * B, B)])
        return carry
    lax.fori_loop(0, ROWS_T // B, _zero, 0)
    plsc.subcore_barrier()

    def _round(r, carry):
        base_e = wid * EPT + r * (RB * B)
        pltpu.sync_copy(gidx_h.at[pl.ds(base_e, RB * B)], gi_r)
        pltpu.sync_copy(w_h.at[pl.ds(base_e, RB * B)], w_r)
        pltpu.sync_copy(dst4_h.at[wid, r], dst_r)

        def _g(m, j):
            pltpu.async_copy(hall_h.at[gi_r.at[pl.ds(m * B, B)]],
                             bufs[j], gsems[j])

        def _gwait(m, j):
            pltpu.make_async_copy(hall_h.at[gi_r.at[pl.ds(m * B, B)]],
                                  bufs[j], gsems[j]).wait()

        def _s(m, j):
            pltpu.async_copy(bufs[j], acc_sh.at[dst_r.at[m]], ssems[j],
                             add=True)

        def _swait(m, j):
            pltpu.make_async_copy(bufs[j], acc_sh.at[dst_r.at[m]],
                                  ssems[j]).wait()

        def _scale(m, buf):
            def _row16(g2, c2):
                wv = w_r[pl.ds(m * B + g2 * 16, 16)]
                base = g2 * 16
                for j in range(16):
                    wj = wv[j]
                    for c8 in range(8):
                        buf[base + j, pl.ds(c8 * 16, 16)] = (
                            buf[base + j, pl.ds(c8 * 16, 16)] * wj)
                return c2
            lax.fori_loop(0, B // 16, _row16, 0)

        _g(0, 0)
        _g(1, 1)
        _g(2, 2)
        _gwait(0, 0); _scale(0, msg0); _s(0, 0)
        _swait(0, 0); _g(3, 0)
        _gwait(1, 1); _scale(1, msg1); _s(1, 1)
        _swait(1, 1); _g(4, 1)
        _gwait(2, 2); _scale(2, msg2); _s(2, 2)

        def _grp(g, c2):
            m0 = 3 + 3 * g
            for i in range(3):
                m = m0 + i
                jn = (i + 2) % 3
                _swait(m - 1, jn)
                _g(m + 2, jn)
                _gwait(m, i)
                _scale(m, bufs[i])
                _s(m, i)
            return c2
        lax.fori_loop(0, 6, _grp, 0)

        _swait(20, 2); _g(23, 2)
        _gwait(21, 0); _scale(21, msg0); _s(21, 0)
        _swait(21, 0); _g(24, 0)
        _gwait(22, 1); _scale(22, msg1); _s(22, 1)
        _gwait(23, 2); _scale(23, msg2); _s(23, 2)
        _gwait(24, 0); _scale(24, msg0); _s(24, 0)
        _swait(22, 1)
        _swait(23, 2)
        _swait(24, 0)
        return carry
    lax.fori_loop(0, NRND, _round, 0)
    plsc.subcore_barrier()

    def _out(k, carry):
        row0 = sid * ROWS_T + k * B
        pltpu.sync_copy(acc_sh.at[pl.ds(row0, B)], msg0)
        pltpu.sync_copy(msg0, parts_h.at[cid, pl.ds(row0, B)])
        return carry
    lax.fori_loop(0, ROWS_T // B, _out, 0)


_layer_sc = pl.kernel(
    _layer_body,
    out_type=_f32((NC, NPAD, HID)),
    mesh=_mesh,
    scratch_types=[
        pltpu.VMEM_SHARED((NPAD, HID), jnp.float32),
        pltpu.VMEM((RB * B,), jnp.int32),
        pltpu.VMEM((RB * B,), jnp.float32),
        pltpu.VMEM((RB, B), jnp.int32),
        pltpu.VMEM((B, HID), jnp.float32),
        pltpu.VMEM((B, HID), jnp.float32),
        pltpu.VMEM((B, HID), jnp.float32),
        pltpu.SemaphoreType.DMA,
        pltpu.SemaphoreType.DMA,
        pltpu.SemaphoreType.DMA,
        pltpu.SemaphoreType.DMA,
        pltpu.SemaphoreType.DMA,
        pltpu.SemaphoreType.DMA,
    ],
)


BN = 1000


def _mm_body(x_ref, w_ref, o_ref):
    o_ref[0] = jnp.dot(x_ref[...], w_ref[0],
                       preferred_element_type=jnp.float32)


def _hall(x, W_rel):
    return pl.pallas_call(
        _mm_body,
        grid=(R, N // BN),
        in_specs=[
            pl.BlockSpec((BN, D), lambda r, n: (n, 0)),
            pl.BlockSpec((1, D, HID), lambda r, n: (r, 0, 0)),
        ],
        out_specs=pl.BlockSpec((1, BN, HID), lambda r, n: (r, n, 0)),
        out_shape=_f32((R, N, HID)),
    )(x, W_rel)


def _comb_hall_body(x_ref, p0_ref, p1_ref, root_ref, b_ref, w2_ref,
                    h_ref, hall_ref):
    agg = p0_ref[...] + p1_ref[...]
    z = agg + jnp.dot(x_ref[...], root_ref[...],
                      preferred_element_type=jnp.float32) + b_ref[...]
    h = jnp.maximum(z, 0.0)
    h_ref[...] = h
    for r in range(R):
        hall_ref[r] = jnp.dot(h, w2_ref[r],
                              preferred_element_type=jnp.float32)


def _comb_hall(x, p0, p1, root, b, W_rel2):
    return pl.pallas_call(
        _comb_hall_body,
        grid=(N // BN,),
        in_specs=[
            pl.BlockSpec((BN, D), lambda n: (n, 0)),
            pl.BlockSpec((BN, HID), lambda n: (n, 0)),
            pl.BlockSpec((BN, HID), lambda n: (n, 0)),
            pl.BlockSpec((D, HID), lambda n: (0, 0)),
            pl.BlockSpec((1, HID), lambda n: (0, 0)),
            pl.BlockSpec((R, HID, HID), lambda n: (0, 0, 0)),
        ],
        out_specs=[
            pl.BlockSpec((BN, HID), lambda n: (n, 0)),
            pl.BlockSpec((R, BN, HID), lambda n: (0, n, 0)),
        ],
        out_shape=[_f32((N, HID)), _f32((R, N, HID))],
    )(x, p0, p1, root, b, W_rel2)


def _comb_pool_body(h1_ref, p0_ref, p1_ref, root_ref, b_ref, batch_ref,
                    linW_ref, linb_ref, o_ref, acc_sc, cnt_sc):
    i = pl.program_id(0)

    @pl.when(i == 0)
    def _init():
        acc_sc[...] = jnp.zeros_like(acc_sc)
        cnt_sc[...] = jnp.zeros_like(cnt_sc)

    agg = p0_ref[...] + p1_ref[...]
    z = agg + jnp.dot(h1_ref[...], root_ref[...],
                      preferred_element_type=jnp.float32) + b_ref[...]
    h2 = jnp.maximum(z, 0.0)

    bvec = batch_ref[0, 0, :]
    gids = lax.broadcasted_iota(jnp.int32, (G, BN), 0)
    oh = (gids == bvec[None, :]).astype(jnp.float32)
    acc_sc[...] += jnp.dot(oh, h2, preferred_element_type=jnp.float32)
    cnt_sc[...] += jnp.broadcast_to(
        jnp.sum(oh, axis=1, keepdims=True), (G, HID))

    @pl.when(i == N // BN - 1)
    def _fin():
        pooled = acc_sc[...] / jnp.maximum(cnt_sc[...], 1.0)
        o_ref[...] = jnp.dot(pooled, linW_ref[...],
                             preferred_element_type=jnp.float32) + linb_ref[...]


def _comb_pool(h1, p0, p1, root, b, batch3, linW, linb):
    return pl.pallas_call(
        _comb_pool_body,
        grid=(N // BN,),
        in_specs=[
            pl.BlockSpec((BN, HID), lambda n: (n, 0)),
            pl.BlockSpec((BN, HID), lambda n: (n, 0)),
            pl.BlockSpec((BN, HID), lambda n: (n, 0)),
            pl.BlockSpec((HID, HID), lambda n: (0, 0)),
            pl.BlockSpec((1, HID), lambda n: (0, 0)),
            pl.BlockSpec((1, 1, BN), lambda n: (n, 0, 0)),
            pl.BlockSpec((HID, C), lambda n: (0, 0)),
            pl.BlockSpec((1, C), lambda n: (0, 0)),
        ],
        out_specs=pl.BlockSpec((G, C), lambda n: (0, 0)),
        out_shape=_f32((G, C)),
        scratch_shapes=[
            pltpu.VMEM((G, HID), jnp.float32),
            pltpu.VMEM((G, HID), jnp.float32),
        ],
    )(h1, p0, p1, root, b, batch3, linW, linb)


def kernel(x, edge_index, edge_type, batch,
           W_rel1, root1, b1, W_rel2, root2, b2, linW, linb):
    src = edge_index[0]
    dst = edge_index[1]
    dst2 = dst.reshape(NW, NRND, RB, B)

    w, gidx = _prep(edge_type, src, dst)

    hall1 = _hall(x, W_rel1).reshape(R * N, HID)
    parts1 = _layer_sc(hall1, gidx, dst2, w)
    h1, hall2 = _comb_hall(x, parts1[0], parts1[1], root1,
                           b1.reshape(1, HID), W_rel2)

    parts2 = _layer_sc(hall2.reshape(R * N, HID), gidx, dst2, w)
    return _comb_pool(h1, parts2[0], parts2[1], root2, b2.reshape(1, HID),
                      batch.reshape(N // BN, 1, BN), linW, linb.reshape(1, C))
